# baseline (speedup 1.0000x reference)
import jax
import jax.numpy as jnp
from jax.experimental import pallas as pl
from jax.experimental.pallas import tpu as pltpu

N_NODES = 10000
D_FEAT = 128
MASK_NUM = 2000


def _attrmask_body(idx_ref, x_ref, out_ref):
    xv = x_ref[...]
    out_ref[...] = xv
    token = jnp.sum(xv, axis=0, keepdims=True) * (1.0 / N_NODES)

    def body(i, tok):
        out_ref[pl.ds(idx_ref[i], 1), :] = tok
        return tok

    jax.lax.fori_loop(0, MASK_NUM, body, token)


def kernel(x, edge_index, mask_idx):
    idx = mask_idx.astype(jnp.int32)
    x_new = pl.pallas_call(
        _attrmask_body,
        out_shape=jax.ShapeDtypeStruct(x.shape, x.dtype),
        in_specs=[
            pl.BlockSpec(memory_space=pltpu.SMEM),
            pl.BlockSpec(memory_space=pltpu.VMEM),
        ],
        out_specs=pl.BlockSpec(memory_space=pltpu.VMEM),
    )(idx, x)
    return (x_new, edge_index)


# speedup vs baseline: 1.8286x; 1.8286x over previous
import jax
import jax.numpy as jnp
from jax.experimental import pallas as pl
from jax.experimental.pallas import tpu as pltpu

N_NODES = 10000
D_FEAT = 128
MASK_NUM = 2000


def _attrmask_body(idx_ref, x_ref, out_ref):
    xv = x_ref[...]
    out_ref[...] = xv
    token = jnp.sum(xv, axis=0, keepdims=True) * (1.0 / N_NODES)

    def body(i, tok):
        out_ref[pl.ds(idx_ref[i], 1), :] = tok
        return tok

    jax.lax.fori_loop(0, MASK_NUM, body, token, unroll=8)


def kernel(x, edge_index, mask_idx):
    idx = mask_idx.astype(jnp.int32)
    x_new = pl.pallas_call(
        _attrmask_body,
        out_shape=jax.ShapeDtypeStruct(x.shape, x.dtype),
        in_specs=[
            pl.BlockSpec(memory_space=pltpu.SMEM),
            pl.BlockSpec(memory_space=pltpu.VMEM),
        ],
        out_specs=pl.BlockSpec(memory_space=pltpu.VMEM),
    )(idx, x)
    return (x_new, edge_index)
